# baseline (speedup 1.0000x reference)
import jax
import jax.numpy as jnp
from jax.experimental import pallas as pl
from jax.experimental.pallas import tpu as pltpu

B, T, D = 1, 2048, 768
H, KVH, DH = 16, 1, 64
G = H // KVH
BLK = 32
NSEL = 16
BALL = 128
W = T // BLK
NB = T // BALL
SCALE = DH ** -0.5
TQ = 256
NQT = T // TQ
NEG = -jnp.finfo(jnp.float32).max / 10.0

_INTERPRET = False


def _dotT(a, b):
    return jax.lax.dot_general(a, b, (((1,), (1,)), ((), ())),
                               preferred_element_type=jnp.float32)


def _dot(a, b):
    return jnp.dot(a, b, preferred_element_type=jnp.float32)


def _gate_col(g_ref):
    return jnp.transpose(g_ref[0])


def _qkv_kernel(x_ref, gamma_ref, wqkv_ref, wcomb_ref, bcomb_ref,
                qkv_ref, gate_ref):
    x = x_ref[...]
    eps = jnp.finfo(jnp.float32).eps
    xn = x * jax.lax.rsqrt(jnp.mean(x * x, axis=-1, keepdims=True) + eps)
    xn = xn * gamma_ref[...]
    qkv_ref[...] = _dot(xn, wqkv_ref[...])
    gate_ref[...] = jax.nn.sigmoid(_dot(xn, wcomb_ref[...]) + bcomb_ref[...])


def _compress_kernel(kw_ref, vw_ref, kW1_ref, kb1_ref, kW2_ref, kb2_ref,
                     vW1_ref, vb1_ref, vW2_ref, vb2_ref, memk_ref, memv_ref,
                     ck_ref, cv_ref):
    hk = jnp.maximum(_dot(kw_ref[...], kW1_ref[...]) + kb1_ref[...], 0.0)
    ck = _dot(hk, kW2_ref[...]) + kb2_ref[...]
    hv = jnp.maximum(_dot(vw_ref[...], vW1_ref[...]) + vb1_ref[...], 0.0)
    cv = _dot(hv, vW2_ref[...]) + vb2_ref[...]
    ck_ref[...] = jnp.zeros((2 * W, DH), jnp.float32)
    cv_ref[...] = jnp.zeros((2 * W, DH), jnp.float32)
    ck_ref[0:W, :] = ck
    cv_ref[0:W, :] = cv
    ck_ref[W:W + 1, :] = memk_ref[...]
    cv_ref[W:W + 1, :] = memv_ref[...]


def _cattn_kernel(q_ref, ck_ref, cv_ref, g_ref, co_ref, imp_ref):
    h = pl.program_id(1)
    q = q_ref[...]
    s = _dotT(q, ck_ref[...]) * SCALE
    col = jax.lax.broadcasted_iota(jnp.int32, s.shape, 1)
    sm = jnp.where(col <= W, s, NEG)
    m = jnp.max(sm, axis=-1, keepdims=True)
    e = jnp.exp(sm - m)
    p = e / jnp.sum(e, axis=-1, keepdims=True)
    co_ref[...] = _dot(p, cv_ref[...]) * _gate_col(g_ref)
    impc = s[:, 0:W]

    @pl.when(h == 0)
    def _():
        imp_ref[...] = impc

    @pl.when(h > 0)
    def _():
        imp_ref[...] = imp_ref[...] + impc

    @pl.when(h == H - 1)
    def _():
        imp_ref[...] = imp_ref[...] * (1.0 / G)


def _mask_kernel(imp_ref, mask_ref):
    imp = imp_ref[...]
    a = imp[:, :, None]
    b = imp[:, None, :]
    i_idx = jax.lax.broadcasted_iota(jnp.int32, (TQ, W, W), 1)
    j_idx = jax.lax.broadcasted_iota(jnp.int32, (TQ, W, W), 2)
    beats = (a > b) | ((a == b) & (i_idx < j_idx))
    rank = jnp.sum(beats.astype(jnp.float32), axis=1)
    mask_ref[...] = (rank < NSEL).astype(jnp.float32)


def _fine_kernel(q_ref, k_ref, v_ref, mask_ref, g_ref, out_ref):
    q = q_ref[...]
    s = _dotT(q, k_ref[...]) * SCALE
    mblk = mask_ref[...]
    r = jax.lax.broadcasted_iota(jnp.int32, (W, T), 0)
    c = jax.lax.broadcasted_iota(jnp.int32, (W, T), 1)
    expand = (c // BLK == r).astype(jnp.float32)
    tok = _dot(mblk, expand)
    sm = jnp.where(tok > 0.5, s, NEG)
    m = jnp.max(sm, axis=-1, keepdims=True)
    e = jnp.exp(sm - m)
    p = e / jnp.sum(e, axis=-1, keepdims=True)
    out_ref[...] = _dot(p, v_ref[...]) * _gate_col(g_ref)


def _ball_kernel(q_ref, k_ref, v_ref, pos_ref, sigma_ref, g_ref, out_ref):
    h = pl.program_id(1)
    p = pos_ref[...]
    diff = p[:, None, :] - p[None, :, :]
    d2 = jnp.sum(diff * diff, axis=-1)
    dist = jnp.sqrt(jnp.maximum(d2, 0.0))
    bias = sigma_ref[h] * dist
    s = _dotT(q_ref[...], k_ref[...]) * SCALE + bias
    m = jnp.max(s, axis=-1, keepdims=True)
    e = jnp.exp(s - m)
    pr = e / jnp.sum(e, axis=-1, keepdims=True)
    out_ref[...] = _dot(pr, v_ref[...]) * _gate_col(g_ref)


def _combine_kernel(c_ref, f_ref, s_ref, wout_ref, out_ref):
    h = pl.program_id(1)
    mixed = c_ref[...] + f_ref[...] + s_ref[...]
    part = _dot(mixed, wout_ref[0])

    @pl.when(h == 0)
    def _():
        out_ref[...] = part

    @pl.when(h > 0)
    def _():
        out_ref[...] = out_ref[...] + part


def kernel(inp, pos, gamma, Wqkv, mem_kv, kW1, kb1, kW2, kb2,
           vW1, vb1, vW2, vb2, sigma_att, Wcomb, bcomb, Wout):
    x = inp.reshape(T, D)
    NQKV = H * DH + 2 * KVH * DH

    qkv, gate = pl.pallas_call(
        _qkv_kernel,
        grid=(NQT,),
        in_specs=[
            pl.BlockSpec((TQ, D), lambda i: (i, 0)),
            pl.BlockSpec((1, D), lambda i: (0, 0)),
            pl.BlockSpec((D, NQKV), lambda i: (0, 0)),
            pl.BlockSpec((D, 3 * H), lambda i: (0, 0)),
            pl.BlockSpec((1, 3 * H), lambda i: (0, 0)),
        ],
        out_specs=[
            pl.BlockSpec((TQ, NQKV), lambda i: (i, 0)),
            pl.BlockSpec((TQ, 3 * H), lambda i: (i, 0)),
        ],
        out_shape=[
            jax.ShapeDtypeStruct((T, NQKV), jnp.float32),
            jax.ShapeDtypeStruct((T, 3 * H), jnp.float32),
        ],
        interpret=_INTERPRET,
    )(x, gamma.reshape(1, D), Wqkv, Wcomb, bcomb.reshape(1, 3 * H))

    k = jax.lax.slice(qkv, (0, H * DH), (T, H * DH + DH))
    v = jax.lax.slice(qkv, (0, H * DH + DH), (T, H * DH + 2 * DH))
    kw = k.reshape(W, BLK * DH)
    vw = v.reshape(W, BLK * DH)
    q3 = qkv[:, :H * DH].reshape(T, H, DH).transpose(1, 0, 2).reshape(H * T, DH)
    gT3 = gate.T.reshape(3 * H, 1, T)

    full = lambda shape: pl.BlockSpec(shape, lambda: tuple(0 for _ in shape))
    ck, cv = pl.pallas_call(
        _compress_kernel,
        in_specs=[
            full((W, BLK * DH)), full((W, BLK * DH)),
            full((BLK * DH, BLK * DH)), full((1, BLK * DH)),
            full((BLK * DH, DH)), full((1, DH)),
            full((BLK * DH, BLK * DH)), full((1, BLK * DH)),
            full((BLK * DH, DH)), full((1, DH)),
            full((1, DH)), full((1, DH)),
        ],
        out_specs=[full((2 * W, DH)), full((2 * W, DH))],
        out_shape=[
            jax.ShapeDtypeStruct((2 * W, DH), jnp.float32),
            jax.ShapeDtypeStruct((2 * W, DH), jnp.float32),
        ],
        interpret=_INTERPRET,
    )(kw, vw, kW1, kb1.reshape(1, -1), kW2, kb2.reshape(1, -1),
      vW1, vb1.reshape(1, -1), vW2, vb2.reshape(1, -1),
      mem_kv[0, 0], mem_kv[1, 0])

    c_out, imp = pl.pallas_call(
        _cattn_kernel,
        grid=(NQT, H),
        in_specs=[
            pl.BlockSpec((TQ, DH), lambda i, h: (h * NQT + i, 0)),
            pl.BlockSpec((2 * W, DH), lambda i, h: (0, 0)),
            pl.BlockSpec((2 * W, DH), lambda i, h: (0, 0)),
            pl.BlockSpec((1, 1, TQ), lambda i, h: (3 * h, 0, i)),
        ],
        out_specs=[
            pl.BlockSpec((TQ, DH), lambda i, h: (h * NQT + i, 0)),
            pl.BlockSpec((TQ, W), lambda i, h: (i, 0)),
        ],
        out_shape=[
            jax.ShapeDtypeStruct((H * T, DH), jnp.float32),
            jax.ShapeDtypeStruct((T, W), jnp.float32),
        ],
        interpret=_INTERPRET,
    )(q3, ck, cv, gT3)

    selmask = pl.pallas_call(
        _mask_kernel,
        grid=(NQT,),
        in_specs=[pl.BlockSpec((TQ, W), lambda i: (i, 0))],
        out_specs=pl.BlockSpec((TQ, W), lambda i: (i, 0)),
        out_shape=jax.ShapeDtypeStruct((T, W), jnp.float32),
        interpret=_INTERPRET,
    )(imp)

    f_out = pl.pallas_call(
        _fine_kernel,
        grid=(NQT, H),
        in_specs=[
            pl.BlockSpec((TQ, DH), lambda i, h: (h * NQT + i, 0)),
            pl.BlockSpec((T, DH), lambda i, h: (0, 0)),
            pl.BlockSpec((T, DH), lambda i, h: (0, 0)),
            pl.BlockSpec((TQ, W), lambda i, h: (i, 0)),
            pl.BlockSpec((1, 1, TQ), lambda i, h: (3 * h + 1, 0, i)),
        ],
        out_specs=pl.BlockSpec((TQ, DH), lambda i, h: (h * NQT + i, 0)),
        out_shape=jax.ShapeDtypeStruct((H * T, DH), jnp.float32),
        interpret=_INTERPRET,
    )(q3, k, v, selmask, gT3)

    posp = jnp.pad(pos, ((0, 0), (0, 8 - pos.shape[1])))
    sigma = sigma_att.reshape(H)
    s_out = pl.pallas_call(
        _ball_kernel,
        grid=(NB, H),
        in_specs=[
            pl.BlockSpec((BALL, DH), lambda b, h: (h * NB + b, 0)),
            pl.BlockSpec((BALL, DH), lambda b, h: (b, 0)),
            pl.BlockSpec((BALL, DH), lambda b, h: (b, 0)),
            pl.BlockSpec((BALL, 8), lambda b, h: (b, 0)),
            pl.BlockSpec(memory_space=pltpu.SMEM),
            pl.BlockSpec((1, 1, BALL), lambda b, h: (3 * h + 2, 0, b)),
        ],
        out_specs=pl.BlockSpec((BALL, DH), lambda b, h: (h * NB + b, 0)),
        out_shape=jax.ShapeDtypeStruct((H * T, DH), jnp.float32),
        interpret=_INTERPRET,
    )(q3, k, v, posp, sigma, gT3)

    Wout3 = Wout.reshape(H, DH, D)
    out = pl.pallas_call(
        _combine_kernel,
        grid=(NQT, H),
        in_specs=[
            pl.BlockSpec((TQ, DH), lambda i, h: (h * NQT + i, 0)),
            pl.BlockSpec((TQ, DH), lambda i, h: (h * NQT + i, 0)),
            pl.BlockSpec((TQ, DH), lambda i, h: (h * NQT + i, 0)),
            pl.BlockSpec((1, DH, D), lambda i, h: (h, 0, 0)),
        ],
        out_specs=pl.BlockSpec((TQ, D), lambda i, h: (i, 0)),
        out_shape=jax.ShapeDtypeStruct((T, D), jnp.float32),
        interpret=_INTERPRET,
    )(c_out, f_out, s_out, Wout3)

    return out.reshape(B, T, D)


# speedup vs baseline: 3.7669x; 3.7669x over previous
import jax
import jax.numpy as jnp
from jax.experimental import pallas as pl
from jax.experimental.pallas import tpu as pltpu

B, T, D = 1, 2048, 768
H, KVH, DH = 16, 1, 64
G = H // KVH
BLK = 32
NSEL = 16
BALL = 128
W = T // BLK
NB = T // BALL
SCALE = DH ** -0.5
TQ = 256
NQT = T // TQ
HB = 4
NEG = -jnp.finfo(jnp.float32).max / 10.0

_INTERPRET = False


def _dotT(a, b):
    return jax.lax.dot_general(a, b, (((1,), (1,)), ((), ())),
                               preferred_element_type=jnp.float32)


def _dot(a, b):
    return jnp.dot(a, b, preferred_element_type=jnp.float32)


def _softmax(s):
    m = jnp.max(s, axis=-1, keepdims=True)
    e = jnp.exp(s - m)
    return e / jnp.sum(e, axis=-1, keepdims=True)


def _softmax_unnorm(s):
    m = jnp.max(s, axis=-1, keepdims=True)
    e = jnp.exp(s - m)
    denom = jnp.sum(e, axis=-1, keepdims=True)
    return e.astype(jnp.bfloat16), 1.0 / denom


def _qkv_kernel(x_ref, gamma_ref, wqkv_ref, wcomb_ref, bcomb_ref,
                qkv_ref, gate_ref):
    x = x_ref[...]
    eps = jnp.finfo(jnp.float32).eps
    xn = x * jax.lax.rsqrt(jnp.mean(x * x, axis=-1, keepdims=True) + eps)
    xn = xn * gamma_ref[...]
    qkv_ref[...] = _dot(xn, wqkv_ref[...])
    gate_ref[...] = jax.nn.sigmoid(_dot(xn, wcomb_ref[...]) + bcomb_ref[...])


def _compress_kernel(kw_ref, vw_ref, kW1_ref, kb1_ref, kW2_ref, kb2_ref,
                     vW1_ref, vb1_ref, vW2_ref, vb2_ref, memk_ref, memv_ref,
                     ck_ref, cv_ref):
    hk = jnp.maximum(_dot(kw_ref[...], kW1_ref[...]) + kb1_ref[...], 0.0)
    ck = _dot(hk, kW2_ref[...]) + kb2_ref[...]
    hv = jnp.maximum(_dot(vw_ref[...], vW1_ref[...]) + vb1_ref[...], 0.0)
    cv = _dot(hv, vW2_ref[...]) + vb2_ref[...]
    ck_ref[...] = jnp.zeros((2 * W, DH), jnp.float32)
    cv_ref[...] = jnp.zeros((2 * W, DH), jnp.float32)
    ck_ref[0:W, :] = ck
    cv_ref[0:W, :] = cv
    ck_ref[W:W + 1, :] = memk_ref[...]
    cv_ref[W:W + 1, :] = memv_ref[...]


def _cattn_kernel(q_ref, ck_ref, cv_ref, g_ref, co_ref, mask_ref):
    q = q_ref[...].reshape(H * TQ, DH)
    s = _dotT(q, ck_ref[...]) * SCALE
    col = jax.lax.broadcasted_iota(jnp.int32, s.shape, 1)
    sm = jnp.where(col <= W, s, NEG)
    e, rdenom = _softmax_unnorm(sm)
    co = _dot(e, cv_ref[...]) * rdenom * g_ref[...].reshape(H * TQ, 1)
    co_ref[...] = co.reshape(H, TQ, DH)
    imp = jnp.mean(s.reshape(H, TQ, 2 * W)[:, :, 0:W], axis=0)
    a = imp[:, :, None]
    b = imp[:, None, :]
    i_idx = jax.lax.broadcasted_iota(jnp.int32, (TQ, W, W), 1)
    j_idx = jax.lax.broadcasted_iota(jnp.int32, (TQ, W, W), 2)
    beats = (a > b) | ((a == b) & (i_idx < j_idx))
    rank = jnp.sum(beats.astype(jnp.float32), axis=1)
    mask_ref[...] = (rank < NSEL).astype(jnp.float32)


def _fine_kernel(q_ref, k_ref, v_ref, mask_ref, g_ref, out_ref):
    q = q_ref[...].reshape(HB * TQ, DH) * SCALE
    s = _dotT(q, k_ref[...])
    mblk = mask_ref[...]
    r = jax.lax.broadcasted_iota(jnp.int32, (W, T), 0)
    c = jax.lax.broadcasted_iota(jnp.int32, (W, T), 1)
    expand = (c // BLK == r).astype(jnp.float32)
    tok = _dot(mblk, expand)
    sm = jnp.where(tok[None] > 0.5, s.reshape(HB, TQ, T), NEG)
    e, rdenom = _softmax_unnorm(sm)
    out = _dot(e.reshape(HB * TQ, T), v_ref[...]).reshape(HB, TQ, DH)
    out_ref[...] = out * rdenom * g_ref[...]


def _ball_kernel(q_ref, k_ref, v_ref, pos_ref, sigma_ref, g_ref, out_ref):
    p = pos_ref[...]
    pt = jnp.transpose(p)
    d2 = jnp.zeros((BALL, BALL), jnp.float32)
    for ci in range(3):
        diff = p[:, ci:ci + 1] - pt[ci:ci + 1, :]
        d2 = d2 + diff * diff
    dist = jnp.sqrt(jnp.maximum(d2, 0.0))
    sig = jnp.broadcast_to(sigma_ref[...], (H, BALL, 1)).reshape(H * BALL, 1)
    bias = sig * jnp.broadcast_to(dist[None], (H, BALL, BALL)
                                  ).reshape(H * BALL, BALL)
    q = q_ref[...].reshape(H * BALL, DH) * SCALE
    s = _dotT(q, k_ref[...]) + bias
    e, rdenom = _softmax_unnorm(s)
    out = _dot(e, v_ref[...]) * rdenom * g_ref[...].reshape(H * BALL, 1)
    out_ref[...] = out.reshape(H, BALL, DH)


def _combine_kernel(c_ref, f_ref, s_ref, wout_ref, out_ref):
    mixed = c_ref[...] + f_ref[...] + s_ref[...]
    acc = _dot(mixed[0], wout_ref[0:DH, :])
    for h in range(1, H):
        acc = acc + _dot(mixed[h], wout_ref[h * DH:(h + 1) * DH, :])
    out_ref[...] = acc


def kernel(inp, pos, gamma, Wqkv, mem_kv, kW1, kb1, kW2, kb2,
           vW1, vb1, vW2, vb2, sigma_att, Wcomb, bcomb, Wout):
    x = inp.reshape(T, D)
    NQKV = H * DH + 2 * KVH * DH

    qkv, gate = pl.pallas_call(
        _qkv_kernel,
        grid=(NQT,),
        in_specs=[
            pl.BlockSpec((TQ, D), lambda i: (i, 0)),
            pl.BlockSpec((1, D), lambda i: (0, 0)),
            pl.BlockSpec((D, NQKV), lambda i: (0, 0)),
            pl.BlockSpec((D, 3 * H), lambda i: (0, 0)),
            pl.BlockSpec((1, 3 * H), lambda i: (0, 0)),
        ],
        out_specs=[
            pl.BlockSpec((TQ, NQKV), lambda i: (i, 0)),
            pl.BlockSpec((TQ, 3 * H), lambda i: (i, 0)),
        ],
        out_shape=[
            jax.ShapeDtypeStruct((T, NQKV), jnp.float32),
            jax.ShapeDtypeStruct((T, 3 * H), jnp.float32),
        ],
        interpret=_INTERPRET,
    )(x, gamma.reshape(1, D), Wqkv, Wcomb, bcomb.reshape(1, 3 * H))

    k = jax.lax.slice(qkv, (0, H * DH), (T, H * DH + DH))
    v = jax.lax.slice(qkv, (0, H * DH + DH), (T, H * DH + 2 * DH))
    kw = k.reshape(W, BLK * DH)
    vw = v.reshape(W, BLK * DH)
    vb = v.astype(jnp.bfloat16)
    q4 = qkv[:, :H * DH].reshape(T, H, DH).transpose(1, 0, 2)
    g0, g1, g2 = (gate[:, j::3].T.reshape(H, T, 1) for j in (0, 1, 2))

    full = lambda shape: pl.BlockSpec(shape, lambda: tuple(0 for _ in shape))
    ck, cv = pl.pallas_call(
        _compress_kernel,
        in_specs=[
            full((W, BLK * DH)), full((W, BLK * DH)),
            full((BLK * DH, BLK * DH)), full((1, BLK * DH)),
            full((BLK * DH, DH)), full((1, DH)),
            full((BLK * DH, BLK * DH)), full((1, BLK * DH)),
            full((BLK * DH, DH)), full((1, DH)),
            full((1, DH)), full((1, DH)),
        ],
        out_specs=[full((2 * W, DH)), full((2 * W, DH))],
        out_shape=[
            jax.ShapeDtypeStruct((2 * W, DH), jnp.float32),
            jax.ShapeDtypeStruct((2 * W, DH), jnp.float32),
        ],
        interpret=_INTERPRET,
    )(kw, vw, kW1, kb1.reshape(1, -1), kW2, kb2.reshape(1, -1),
      vW1, vb1.reshape(1, -1), vW2, vb2.reshape(1, -1),
      mem_kv[0, 0], mem_kv[1, 0])

    c4, selmask = pl.pallas_call(
        _cattn_kernel,
        grid=(NQT,),
        in_specs=[
            pl.BlockSpec((H, TQ, DH), lambda i: (0, i, 0)),
            pl.BlockSpec((2 * W, DH), lambda i: (0, 0)),
            pl.BlockSpec((2 * W, DH), lambda i: (0, 0)),
            pl.BlockSpec((H, TQ, 1), lambda i: (0, i, 0)),
        ],
        out_specs=[
            pl.BlockSpec((H, TQ, DH), lambda i: (0, i, 0)),
            pl.BlockSpec((TQ, W), lambda i: (i, 0)),
        ],
        out_shape=[
            jax.ShapeDtypeStruct((H, T, DH), jnp.float32),
            jax.ShapeDtypeStruct((T, W), jnp.float32),
        ],
        interpret=_INTERPRET,
    )(q4, ck, cv.astype(jnp.bfloat16), g0)

    f4 = pl.pallas_call(
        _fine_kernel,
        grid=(NQT, H // HB),
        in_specs=[
            pl.BlockSpec((HB, TQ, DH), lambda i, h: (h, i, 0)),
            pl.BlockSpec((T, DH), lambda i, h: (0, 0)),
            pl.BlockSpec((T, DH), lambda i, h: (0, 0)),
            pl.BlockSpec((TQ, W), lambda i, h: (i, 0)),
            pl.BlockSpec((HB, TQ, 1), lambda i, h: (h, i, 0)),
        ],
        out_specs=pl.BlockSpec((HB, TQ, DH), lambda i, h: (h, i, 0)),
        out_shape=jax.ShapeDtypeStruct((H, T, DH), jnp.float32),
        interpret=_INTERPRET,
    )(q4, k, vb, selmask, g1)

    posp = jnp.pad(pos, ((0, 0), (0, 8 - pos.shape[1])))
    sigma3 = sigma_att.reshape(H, 1, 1)
    s4 = pl.pallas_call(
        _ball_kernel,
        grid=(NB,),
        in_specs=[
            pl.BlockSpec((H, BALL, DH), lambda b: (0, b, 0)),
            pl.BlockSpec((BALL, DH), lambda b: (b, 0)),
            pl.BlockSpec((BALL, DH), lambda b: (b, 0)),
            pl.BlockSpec((BALL, 8), lambda b: (b, 0)),
            pl.BlockSpec((H, 1, 1), lambda b: (0, 0, 0)),
            pl.BlockSpec((H, BALL, 1), lambda b: (0, b, 0)),
        ],
        out_specs=pl.BlockSpec((H, BALL, DH), lambda b: (0, b, 0)),
        out_shape=jax.ShapeDtypeStruct((H, T, DH), jnp.float32),
        interpret=_INTERPRET,
    )(q4, k, vb, posp, sigma3, g2)

    out = pl.pallas_call(
        _combine_kernel,
        grid=(NQT,),
        in_specs=[
            pl.BlockSpec((H, TQ, DH), lambda i: (0, i, 0)),
            pl.BlockSpec((H, TQ, DH), lambda i: (0, i, 0)),
            pl.BlockSpec((H, TQ, DH), lambda i: (0, i, 0)),
            pl.BlockSpec((H * DH, D), lambda i: (0, 0)),
        ],
        out_specs=pl.BlockSpec((TQ, D), lambda i: (i, 0)),
        out_shape=jax.ShapeDtypeStruct((T, D), jnp.float32),
        interpret=_INTERPRET,
    )(c4, f4, s4, Wout)

    return out.reshape(B, T, D)


# speedup vs baseline: 4.8237x; 1.2805x over previous
import jax
import jax.numpy as jnp
from jax.experimental import pallas as pl
from jax.experimental.pallas import tpu as pltpu

B, T, D = 1, 2048, 768
H, KVH, DH = 16, 1, 64
G = H // KVH
BLK = 32
NSEL = 16
BALL = 128
W = T // BLK
NB = T // BALL
SCALE = DH ** -0.5
TQ = 256
NQT = T // TQ
HB = 4
NEG = -jnp.finfo(jnp.float32).max / 10.0

_INTERPRET = False


def _dotT(a, b):
    return jax.lax.dot_general(a, b, (((1,), (1,)), ((), ())),
                               preferred_element_type=jnp.float32)


def _dot(a, b):
    return jnp.dot(a, b, preferred_element_type=jnp.float32)


def _softmax(s):
    m = jnp.max(s, axis=-1, keepdims=True)
    e = jnp.exp(s - m)
    return e / jnp.sum(e, axis=-1, keepdims=True)


def _softmax_unnorm(s):
    m = jnp.max(s, axis=-1, keepdims=True)
    e = jnp.exp(s - m)
    denom = jnp.sum(e, axis=-1, keepdims=True)
    return e.astype(jnp.bfloat16), 1.0 / denom


def _qkv_kernel(x_ref, gamma_ref, wqkv_ref, wcomb_ref, bcomb_ref,
                qkv_ref, gate_ref):
    x = x_ref[...]
    eps = jnp.finfo(jnp.float32).eps
    xn = x * jax.lax.rsqrt(jnp.mean(x * x, axis=-1, keepdims=True) + eps)
    xn = xn * gamma_ref[...]
    qkv_ref[...] = _dot(xn, wqkv_ref[...])
    gate_ref[...] = jax.nn.sigmoid(_dot(xn, wcomb_ref[...]) + bcomb_ref[...])


def _compress_kernel(kw_ref, vw_ref, kW1_ref, kb1_ref, kW2_ref, kb2_ref,
                     vW1_ref, vb1_ref, vW2_ref, vb2_ref, memk_ref, memv_ref,
                     ck_ref, cv_ref):
    hk = jnp.maximum(_dot(kw_ref[...], kW1_ref[...]) + kb1_ref[...], 0.0)
    ck = _dot(hk, kW2_ref[...]) + kb2_ref[...]
    hv = jnp.maximum(_dot(vw_ref[...], vW1_ref[...]) + vb1_ref[...], 0.0)
    cv = _dot(hv, vW2_ref[...]) + vb2_ref[...]
    ck_ref[...] = jnp.zeros((2 * W, DH), jnp.float32)
    cv_ref[...] = jnp.zeros((2 * W, DH), jnp.float32)
    ck_ref[0:W, :] = ck
    cv_ref[0:W, :] = cv
    ck_ref[W:W + 1, :] = memk_ref[...]
    cv_ref[W:W + 1, :] = memv_ref[...]


def _cattn_kernel(q_ref, ck_ref, cv_ref, g_ref, co_ref, mask_ref):
    q = q_ref[...].reshape(H * TQ, DH)
    s = _dotT(q, ck_ref[...]) * SCALE
    col = jax.lax.broadcasted_iota(jnp.int32, s.shape, 1)
    sm = jnp.where(col <= W, s, NEG)
    e, rdenom = _softmax_unnorm(sm)
    co = _dot(e, cv_ref[...]) * rdenom * g_ref[...].reshape(H * TQ, 1)
    co_ref[...] = co.reshape(H, TQ, DH)
    imp = jnp.mean(s.reshape(H, TQ, 2 * W)[:, :, 0:W], axis=0)
    a = imp[:, :, None]
    b = imp[:, None, :]
    i_idx = jax.lax.broadcasted_iota(jnp.int32, (TQ, W, W), 1)
    j_idx = jax.lax.broadcasted_iota(jnp.int32, (TQ, W, W), 2)
    beats = (a > b) | ((a == b) & (i_idx < j_idx))
    rank = jnp.sum(beats.astype(jnp.float32), axis=1)
    mask_ref[...] = (rank < NSEL).astype(jnp.float32)


def _fine_kernel(q_ref, k_ref, v_ref, mask_ref, g_ref, c_ref, s_ref,
                 wout_ref, out_ref):
    hg = pl.program_id(1)
    q = q_ref[...].reshape(HB * TQ, DH) * SCALE
    s = _dotT(q, k_ref[...])
    mblk = mask_ref[...]
    r = jax.lax.broadcasted_iota(jnp.int32, (W, T), 0)
    c = jax.lax.broadcasted_iota(jnp.int32, (W, T), 1)
    expand = (c // BLK == r).astype(jnp.float32)
    tok = _dot(mblk, expand)
    sm = jnp.where(tok[None] > 0.5, s.reshape(HB, TQ, T), NEG)
    e, rdenom = _softmax_unnorm(sm)
    fine = _dot(e.reshape(HB * TQ, T), v_ref[...]).reshape(HB, TQ, DH)
    mixed = fine * rdenom * g_ref[...] + c_ref[...] + s_ref[...]
    part = _dot(mixed[0], wout_ref[pl.ds(hg * (HB * DH), DH), :])
    for j in range(1, HB):
        part = part + _dot(
            mixed[j], wout_ref[pl.ds(hg * (HB * DH) + j * DH, DH), :])

    @pl.when(hg == 0)
    def _():
        out_ref[...] = part

    @pl.when(hg > 0)
    def _():
        out_ref[...] = out_ref[...] + part


def _ball_kernel(q_ref, k_ref, v_ref, pos_ref, sigma_ref, g_ref, out_ref):
    p = pos_ref[...]
    pt = jnp.transpose(p)
    d2 = jnp.zeros((BALL, BALL), jnp.float32)
    for ci in range(3):
        diff = p[:, ci:ci + 1] - pt[ci:ci + 1, :]
        d2 = d2 + diff * diff
    dist = jnp.sqrt(jnp.maximum(d2, 0.0))
    sig = jnp.broadcast_to(sigma_ref[...], (H, BALL, 1)).reshape(H * BALL, 1)
    bias = sig * jnp.broadcast_to(dist[None], (H, BALL, BALL)
                                  ).reshape(H * BALL, BALL)
    q = q_ref[...].reshape(H * BALL, DH) * SCALE
    s = _dotT(q, k_ref[...]) + bias
    e, rdenom = _softmax_unnorm(s)
    out = _dot(e, v_ref[...]) * rdenom * g_ref[...].reshape(H * BALL, 1)
    out_ref[...] = out.reshape(H, BALL, DH)


def kernel(inp, pos, gamma, Wqkv, mem_kv, kW1, kb1, kW2, kb2,
           vW1, vb1, vW2, vb2, sigma_att, Wcomb, bcomb, Wout):
    x = inp.reshape(T, D)
    NQKV = H * DH + 2 * KVH * DH

    qkv, gate = pl.pallas_call(
        _qkv_kernel,
        grid=(NQT,),
        in_specs=[
            pl.BlockSpec((TQ, D), lambda i: (i, 0)),
            pl.BlockSpec((1, D), lambda i: (0, 0)),
            pl.BlockSpec((D, NQKV), lambda i: (0, 0)),
            pl.BlockSpec((D, 3 * H), lambda i: (0, 0)),
            pl.BlockSpec((1, 3 * H), lambda i: (0, 0)),
        ],
        out_specs=[
            pl.BlockSpec((TQ, NQKV), lambda i: (i, 0)),
            pl.BlockSpec((TQ, 3 * H), lambda i: (i, 0)),
        ],
        out_shape=[
            jax.ShapeDtypeStruct((T, NQKV), jnp.float32),
            jax.ShapeDtypeStruct((T, 3 * H), jnp.float32),
        ],
        interpret=_INTERPRET,
    )(x, gamma.reshape(1, D), Wqkv, Wcomb, bcomb.reshape(1, 3 * H))

    k = jax.lax.slice(qkv, (0, H * DH), (T, H * DH + DH))
    v = jax.lax.slice(qkv, (0, H * DH + DH), (T, H * DH + 2 * DH))
    kw = k.reshape(W, BLK * DH)
    vw = v.reshape(W, BLK * DH)
    vb = v.astype(jnp.bfloat16)
    q4 = qkv[:, :H * DH].reshape(T, H, DH).transpose(1, 0, 2)
    g0, g1, g2 = (gate[:, j::3].T.reshape(H, T, 1) for j in (0, 1, 2))

    full = lambda shape: pl.BlockSpec(shape, lambda: tuple(0 for _ in shape))
    ck, cv = pl.pallas_call(
        _compress_kernel,
        in_specs=[
            full((W, BLK * DH)), full((W, BLK * DH)),
            full((BLK * DH, BLK * DH)), full((1, BLK * DH)),
            full((BLK * DH, DH)), full((1, DH)),
            full((BLK * DH, BLK * DH)), full((1, BLK * DH)),
            full((BLK * DH, DH)), full((1, DH)),
            full((1, DH)), full((1, DH)),
        ],
        out_specs=[full((2 * W, DH)), full((2 * W, DH))],
        out_shape=[
            jax.ShapeDtypeStruct((2 * W, DH), jnp.float32),
            jax.ShapeDtypeStruct((2 * W, DH), jnp.float32),
        ],
        interpret=_INTERPRET,
    )(kw, vw, kW1, kb1.reshape(1, -1), kW2, kb2.reshape(1, -1),
      vW1, vb1.reshape(1, -1), vW2, vb2.reshape(1, -1),
      mem_kv[0, 0], mem_kv[1, 0])

    c4, selmask = pl.pallas_call(
        _cattn_kernel,
        grid=(NQT,),
        in_specs=[
            pl.BlockSpec((H, TQ, DH), lambda i: (0, i, 0)),
            pl.BlockSpec((2 * W, DH), lambda i: (0, 0)),
            pl.BlockSpec((2 * W, DH), lambda i: (0, 0)),
            pl.BlockSpec((H, TQ, 1), lambda i: (0, i, 0)),
        ],
        out_specs=[
            pl.BlockSpec((H, TQ, DH), lambda i: (0, i, 0)),
            pl.BlockSpec((TQ, W), lambda i: (i, 0)),
        ],
        out_shape=[
            jax.ShapeDtypeStruct((H, T, DH), jnp.float32),
            jax.ShapeDtypeStruct((T, W), jnp.float32),
        ],
        interpret=_INTERPRET,
    )(q4, ck, cv.astype(jnp.bfloat16), g0)

    posp = jnp.pad(pos, ((0, 0), (0, 8 - pos.shape[1])))

    sigma3 = sigma_att.reshape(H, 1, 1)
    s4 = pl.pallas_call(
        _ball_kernel,
        grid=(NB,),
        in_specs=[
            pl.BlockSpec((H, BALL, DH), lambda b: (0, b, 0)),
            pl.BlockSpec((BALL, DH), lambda b: (b, 0)),
            pl.BlockSpec((BALL, DH), lambda b: (b, 0)),
            pl.BlockSpec((BALL, 8), lambda b: (b, 0)),
            pl.BlockSpec((H, 1, 1), lambda b: (0, 0, 0)),
            pl.BlockSpec((H, BALL, 1), lambda b: (0, b, 0)),
        ],
        out_specs=pl.BlockSpec((H, BALL, DH), lambda b: (0, b, 0)),
        out_shape=jax.ShapeDtypeStruct((H, T, DH), jnp.float32),
        interpret=_INTERPRET,
    )(q4, k, vb, posp, sigma3, g2)

    out = pl.pallas_call(
        _fine_kernel,
        grid=(NQT, H // HB),
        in_specs=[
            pl.BlockSpec((HB, TQ, DH), lambda i, h: (h, i, 0)),
            pl.BlockSpec((T, DH), lambda i, h: (0, 0)),
            pl.BlockSpec((T, DH), lambda i, h: (0, 0)),
            pl.BlockSpec((TQ, W), lambda i, h: (i, 0)),
            pl.BlockSpec((HB, TQ, 1), lambda i, h: (h, i, 0)),
            pl.BlockSpec((HB, TQ, DH), lambda i, h: (h, i, 0)),
            pl.BlockSpec((HB, TQ, DH), lambda i, h: (h, i, 0)),
            pl.BlockSpec((H * DH, D), lambda i, h: (0, 0)),
        ],
        out_specs=pl.BlockSpec((TQ, D), lambda i, h: (i, 0)),
        out_shape=jax.ShapeDtypeStruct((T, D), jnp.float32),
        interpret=_INTERPRET,
    )(q4, k, vb, selmask, g1, c4, s4, Wout)

    return out.reshape(B, T, D)


# speedup vs baseline: 4.8408x; 1.0036x over previous
import jax
import jax.numpy as jnp
from jax.experimental import pallas as pl
from jax.experimental.pallas import tpu as pltpu

B, T, D = 1, 2048, 768
H, KVH, DH = 16, 1, 64
G = H // KVH
BLK = 32
NSEL = 16
BALL = 128
W = T // BLK
NB = T // BALL
SCALE = DH ** -0.5
TQ = 256
NQT = T // TQ
HB = 4
NBT = TQ // BALL
NEG = -jnp.finfo(jnp.float32).max / 10.0

_INTERPRET = False


def _dotT(a, b):
    return jax.lax.dot_general(a, b, (((1,), (1,)), ((), ())),
                               preferred_element_type=jnp.float32)


def _dot(a, b):
    return jnp.dot(a, b, preferred_element_type=jnp.float32)


def _softmax_unnorm(s):
    m = jnp.max(s, axis=-1, keepdims=True)
    e = jnp.exp(s - m)
    denom = jnp.sum(e, axis=-1, keepdims=True)
    return e.astype(jnp.bfloat16), 1.0 / denom


def _qkv_kernel(x_ref, gamma_ref, wqkv_ref, wcomb_ref, bcomb_ref,
                qkv_ref, gate_ref):
    x = x_ref[...]
    eps = jnp.finfo(jnp.float32).eps
    xn = x * jax.lax.rsqrt(jnp.mean(x * x, axis=-1, keepdims=True) + eps)
    xn = xn * gamma_ref[...]
    qkv_ref[...] = _dot(xn, wqkv_ref[...])
    gate_ref[...] = jax.nn.sigmoid(_dot(xn, wcomb_ref[...]) + bcomb_ref[...])


def _compress_kernel(kw_ref, vw_ref, kW1_ref, kb1_ref, kW2_ref, kb2_ref,
                     vW1_ref, vb1_ref, vW2_ref, vb2_ref, memk_ref, memv_ref,
                     ck_ref, cv_ref):
    hk = jnp.maximum(_dot(kw_ref[...], kW1_ref[...]) + kb1_ref[...], 0.0)
    ck = _dot(hk, kW2_ref[...]) + kb2_ref[...]
    hv = jnp.maximum(_dot(vw_ref[...], vW1_ref[...]) + vb1_ref[...], 0.0)
    cv = _dot(hv, vW2_ref[...]) + vb2_ref[...]
    ck_ref[...] = jnp.zeros((2 * W, DH), jnp.float32)
    cv_ref[...] = jnp.zeros((2 * W, DH), jnp.float32)
    ck_ref[0:W, :] = ck
    cv_ref[0:W, :] = cv
    ck_ref[W:W + 1, :] = memk_ref[...]
    cv_ref[W:W + 1, :] = memv_ref[...]


def _mega_kernel(q_ref, ck_ref, cv_ref, k_ref, vb_ref, kloc_ref, vloc_ref,
                 pos_ref, sigma_ref, g0_ref, g1_ref, g2_ref, wout_ref,
                 out_ref):
    q_all = q_ref[...]
    q = q_all.reshape(H * TQ, DH)

    s = _dotT(q, ck_ref[...]) * SCALE
    col = jax.lax.broadcasted_iota(jnp.int32, s.shape, 1)
    sm = jnp.where(col <= W, s, NEG)
    e, rdenom = _softmax_unnorm(sm)
    co = (_dot(e, cv_ref[...]) * rdenom * g0_ref[...].reshape(H * TQ, 1)
          ).reshape(H, TQ, DH)
    imp = jnp.mean(s.reshape(H, TQ, 2 * W)[:, :, 0:W], axis=0)
    a = imp[:, :, None]
    bt = imp[:, None, :]
    i_idx = jax.lax.broadcasted_iota(jnp.int32, (TQ, W, W), 1)
    j_idx = jax.lax.broadcasted_iota(jnp.int32, (TQ, W, W), 2)
    beats = (a > bt) | ((a == bt) & (i_idx < j_idx))
    rank = jnp.sum(beats.astype(jnp.float32), axis=1)
    mblk = (rank < NSEL).astype(jnp.float32)
    r = jax.lax.broadcasted_iota(jnp.int32, (W, T), 0)
    c = jax.lax.broadcasted_iota(jnp.int32, (W, T), 1)
    expand = (c // BLK == r).astype(jnp.float32)
    tok = _dot(mblk, expand)

    pos = pos_ref[...]
    g2 = g2_ref[...]
    ball_outs = []
    for bi in range(NBT):
        pb = pos[bi * BALL:(bi + 1) * BALL]
        pt = jnp.transpose(pb)
        d2 = jnp.zeros((BALL, BALL), jnp.float32)
        for ci in range(3):
            diff = pb[:, ci:ci + 1] - pt[ci:ci + 1, :]
            d2 = d2 + diff * diff
        dist = jnp.sqrt(jnp.maximum(d2, 0.0))
        sig = jnp.broadcast_to(sigma_ref[...], (H, BALL, 1)
                               ).reshape(H * BALL, 1)
        bias = sig * jnp.broadcast_to(dist[None], (H, BALL, BALL)
                                      ).reshape(H * BALL, BALL)
        qb = q_all[:, bi * BALL:(bi + 1) * BALL, :].reshape(H * BALL, DH)
        sb = _dotT(qb * SCALE, kloc_ref[bi * BALL:(bi + 1) * BALL, :]) + bias
        eb, rdb = _softmax_unnorm(sb)
        ob = _dot(eb, vloc_ref[bi * BALL:(bi + 1) * BALL, :]) * rdb
        ob = ob * g2[:, bi * BALL:(bi + 1) * BALL, :].reshape(H * BALL, 1)
        ball_outs.append(ob.reshape(H, BALL, DH))
    so = jnp.concatenate(ball_outs, axis=1)

    g1 = g1_ref[...]
    acc = jnp.zeros((TQ, D), jnp.float32)
    for hg in range(H // HB):
        qh = q_all[hg * HB:(hg + 1) * HB].reshape(HB * TQ, DH) * SCALE
        sf = _dotT(qh, k_ref[...])
        smf = jnp.where(tok[None] > 0.5, sf.reshape(HB, TQ, T), NEG)
        ef, rdf = _softmax_unnorm(smf)
        fine = _dot(ef.reshape(HB * TQ, T), vb_ref[...]).reshape(HB, TQ, DH)
        mixed = (fine * rdf * g1[hg * HB:(hg + 1) * HB]
                 + co[hg * HB:(hg + 1) * HB] + so[hg * HB:(hg + 1) * HB])
        for j in range(HB):
            hh = hg * HB + j
            acc = acc + _dot(mixed[j], wout_ref[hh * DH:(hh + 1) * DH, :])
    out_ref[...] = acc


def kernel(inp, pos, gamma, Wqkv, mem_kv, kW1, kb1, kW2, kb2,
           vW1, vb1, vW2, vb2, sigma_att, Wcomb, bcomb, Wout):
    x = inp.reshape(T, D)
    NQKV = H * DH + 2 * KVH * DH

    qkv, gate = pl.pallas_call(
        _qkv_kernel,
        grid=(NQT,),
        in_specs=[
            pl.BlockSpec((TQ, D), lambda i: (i, 0)),
            pl.BlockSpec((1, D), lambda i: (0, 0)),
            pl.BlockSpec((D, NQKV), lambda i: (0, 0)),
            pl.BlockSpec((D, 3 * H), lambda i: (0, 0)),
            pl.BlockSpec((1, 3 * H), lambda i: (0, 0)),
        ],
        out_specs=[
            pl.BlockSpec((TQ, NQKV), lambda i: (i, 0)),
            pl.BlockSpec((TQ, 3 * H), lambda i: (i, 0)),
        ],
        out_shape=[
            jax.ShapeDtypeStruct((T, NQKV), jnp.float32),
            jax.ShapeDtypeStruct((T, 3 * H), jnp.float32),
        ],
        interpret=_INTERPRET,
    )(x, gamma.reshape(1, D), Wqkv, Wcomb, bcomb.reshape(1, 3 * H))

    k = jax.lax.slice(qkv, (0, H * DH), (T, H * DH + DH))
    v = jax.lax.slice(qkv, (0, H * DH + DH), (T, H * DH + 2 * DH))
    kw = k.reshape(W, BLK * DH)
    vw = v.reshape(W, BLK * DH)
    vb = v.astype(jnp.bfloat16)
    q4 = qkv[:, :H * DH].reshape(T, H, DH).transpose(1, 0, 2)
    g0, g1, g2 = (gate[:, j::3].T.reshape(H, T, 1) for j in (0, 1, 2))

    full = lambda shape: pl.BlockSpec(shape, lambda: tuple(0 for _ in shape))
    ck, cv = pl.pallas_call(
        _compress_kernel,
        in_specs=[
            full((W, BLK * DH)), full((W, BLK * DH)),
            full((BLK * DH, BLK * DH)), full((1, BLK * DH)),
            full((BLK * DH, DH)), full((1, DH)),
            full((BLK * DH, BLK * DH)), full((1, BLK * DH)),
            full((BLK * DH, DH)), full((1, DH)),
            full((1, DH)), full((1, DH)),
        ],
        out_specs=[full((2 * W, DH)), full((2 * W, DH))],
        out_shape=[
            jax.ShapeDtypeStruct((2 * W, DH), jnp.float32),
            jax.ShapeDtypeStruct((2 * W, DH), jnp.float32),
        ],
        interpret=_INTERPRET,
    )(kw, vw, kW1, kb1.reshape(1, -1), kW2, kb2.reshape(1, -1),
      vW1, vb1.reshape(1, -1), vW2, vb2.reshape(1, -1),
      mem_kv[0, 0], mem_kv[1, 0])

    posp = jnp.pad(pos, ((0, 0), (0, 8 - pos.shape[1])))
    sigma3 = sigma_att.reshape(H, 1, 1)
    out = pl.pallas_call(
        _mega_kernel,
        grid=(NQT,),
        in_specs=[
            pl.BlockSpec((H, TQ, DH), lambda i: (0, i, 0)),
            pl.BlockSpec((2 * W, DH), lambda i: (0, 0)),
            pl.BlockSpec((2 * W, DH), lambda i: (0, 0)),
            pl.BlockSpec((T, DH), lambda i: (0, 0)),
            pl.BlockSpec((T, DH), lambda i: (0, 0)),
            pl.BlockSpec((TQ, DH), lambda i: (i, 0)),
            pl.BlockSpec((TQ, DH), lambda i: (i, 0)),
            pl.BlockSpec((TQ, 8), lambda i: (i, 0)),
            pl.BlockSpec((H, 1, 1), lambda i: (0, 0, 0)),
            pl.BlockSpec((H, TQ, 1), lambda i: (0, i, 0)),
            pl.BlockSpec((H, TQ, 1), lambda i: (0, i, 0)),
            pl.BlockSpec((H, TQ, 1), lambda i: (0, i, 0)),
            pl.BlockSpec((H * DH, D), lambda i: (0, 0)),
        ],
        out_specs=pl.BlockSpec((TQ, D), lambda i: (i, 0)),
        out_shape=jax.ShapeDtypeStruct((T, D), jnp.float32),
        interpret=_INTERPRET,
    )(q4, ck, cv.astype(jnp.bfloat16), k, vb, k, vb, posp, sigma3,
      g0, g1, g2, Wout)

    return out.reshape(B, T, D)


# speedup vs baseline: 5.2770x; 1.0901x over previous
import jax
import jax.numpy as jnp
from jax.experimental import pallas as pl
from jax.experimental.pallas import tpu as pltpu

B, T, D = 1, 2048, 768
H, KVH, DH = 16, 1, 64
G = H // KVH
BLK = 32
NSEL = 16
BALL = 128
W = T // BLK
NB = T // BALL
SCALE = DH ** -0.5
TQ = 256
NQT = T // TQ
HB = 4
NBT = TQ // BALL
NEG = -jnp.finfo(jnp.float32).max / 10.0
MBIG = 512.0

_INTERPRET = False


def _dotT(a, b):
    return jax.lax.dot_general(a, b, (((1,), (1,)), ((), ())),
                               preferred_element_type=jnp.float32)


def _dot(a, b):
    return jnp.dot(a, b, preferred_element_type=jnp.float32)


def _softmax_unnorm(s):
    m = jnp.max(s, axis=-1, keepdims=True)
    e = jnp.exp(s - m)
    denom = jnp.sum(e, axis=-1, keepdims=True)
    return e.astype(jnp.bfloat16), 1.0 / denom


def _qkv_kernel(x_ref, gamma_ref, wqkv_ref, wcomb_ref, bcomb_ref,
                qkv_ref, gate_ref):
    x = x_ref[...]
    eps = jnp.finfo(jnp.float32).eps
    xn = x * jax.lax.rsqrt(jnp.mean(x * x, axis=-1, keepdims=True) + eps)
    xn = xn * gamma_ref[...]
    qkv_ref[...] = _dot(xn, wqkv_ref[...])
    gate_ref[...] = jax.nn.sigmoid(_dot(xn, wcomb_ref[...]) + bcomb_ref[...])


def _compress_kernel(kw_ref, vw_ref, kW1_ref, kb1_ref, kW2_ref, kb2_ref,
                     vW1_ref, vb1_ref, vW2_ref, vb2_ref, memk_ref, memv_ref,
                     ck_ref, cv_ref):
    hk = jnp.maximum(_dot(kw_ref[...], kW1_ref[...]) + kb1_ref[...], 0.0)
    ck = _dot(hk, kW2_ref[...]) + kb2_ref[...]
    hv = jnp.maximum(_dot(vw_ref[...], vW1_ref[...]) + vb1_ref[...], 0.0)
    cv = _dot(hv, vW2_ref[...]) + vb2_ref[...]
    ck_ref[...] = jnp.zeros((2 * W, DH), jnp.float32)
    cv_ref[...] = jnp.zeros((2 * W, DH), jnp.float32)
    ck_ref[0:W, :] = ck
    cv_ref[0:W, :] = cv
    ck_ref[W:W + 1, :] = memk_ref[...]
    cv_ref[W:W + 1, :] = memv_ref[...]


def _mega_kernel(q_ref, ck_ref, cv_ref, kx_ref, vb_ref, kloc_ref, vloc_ref,
                 pos_ref, sigma_ref, g0_ref, g1_ref, g2_ref, wout_ref,
                 out_ref):
    q_all = q_ref[...]
    q = q_all.reshape(H * TQ, DH)

    s = _dotT(q, ck_ref[...]) * SCALE
    col = jax.lax.broadcasted_iota(jnp.int32, s.shape, 1)
    sm = jnp.where(col <= W, s, NEG)
    e, rdenom = _softmax_unnorm(sm)
    co = (_dot(e, cv_ref[...]) * rdenom * g0_ref[...].reshape(H * TQ, 1)
          ).reshape(H, TQ, DH)
    imp = jnp.mean(s.reshape(H, TQ, 2 * W)[:, :, 0:W], axis=0)
    a = imp[:, :, None]
    bt = imp[:, None, :]
    i_idx = jax.lax.broadcasted_iota(jnp.int32, (TQ, W, W), 1)
    j_idx = jax.lax.broadcasted_iota(jnp.int32, (TQ, W, W), 2)
    beats = (a > bt) | ((a == bt) & (i_idx < j_idx))
    rank = jnp.sum(beats.astype(jnp.float32), axis=1)
    mtile = (rank < NSEL).astype(jnp.float32) * MBIG

    pos = pos_ref[...]
    g2 = g2_ref[...]
    ball_outs = []
    for bi in range(NBT):
        pb = pos[bi * BALL:(bi + 1) * BALL]
        pt = jnp.transpose(pb)
        d2 = jnp.zeros((BALL, BALL), jnp.float32)
        for ci in range(3):
            diff = pb[:, ci:ci + 1] - pt[ci:ci + 1, :]
            d2 = d2 + diff * diff
        dist = jnp.sqrt(jnp.maximum(d2, 0.0))
        sig = jnp.broadcast_to(sigma_ref[...], (H, BALL, 1)
                               ).reshape(H * BALL, 1)
        bias = sig * jnp.broadcast_to(dist[None], (H, BALL, BALL)
                                      ).reshape(H * BALL, BALL)
        qb = q_all[:, bi * BALL:(bi + 1) * BALL, :].reshape(H * BALL, DH)
        sb = _dotT(qb * SCALE, kloc_ref[bi * BALL:(bi + 1) * BALL, :]) + bias
        eb, rdb = _softmax_unnorm(sb)
        ob = _dot(eb, vloc_ref[bi * BALL:(bi + 1) * BALL, :]) * rdb
        ob = ob * g2[:, bi * BALL:(bi + 1) * BALL, :].reshape(H * BALL, 1)
        ball_outs.append(ob.reshape(H, BALL, DH))
    so = jnp.concatenate(ball_outs, axis=1)

    g1 = g1_ref[...]
    acc = jnp.zeros((TQ, D), jnp.float32)
    for hg in range(H // HB):
        qh = q_all[hg * HB:(hg + 1) * HB].reshape(HB * TQ, DH) * SCALE
        qa = jnp.concatenate(
            [qh, jnp.broadcast_to(mtile[None], (HB, TQ, W)
                                  ).reshape(HB * TQ, W)], axis=1)
        sf = _dotT(qa, kx_ref[...])
        ef, rdf = _softmax_unnorm(sf.reshape(HB, TQ, T))
        fine = _dot(ef.reshape(HB * TQ, T), vb_ref[...]).reshape(HB, TQ, DH)
        mixed = (fine * rdf * g1[hg * HB:(hg + 1) * HB]
                 + co[hg * HB:(hg + 1) * HB] + so[hg * HB:(hg + 1) * HB])
        for j in range(HB):
            hh = hg * HB + j
            acc = acc + _dot(mixed[j], wout_ref[hh * DH:(hh + 1) * DH, :])
    out_ref[...] = acc


def kernel(inp, pos, gamma, Wqkv, mem_kv, kW1, kb1, kW2, kb2,
           vW1, vb1, vW2, vb2, sigma_att, Wcomb, bcomb, Wout):
    x = inp.reshape(T, D)
    NQKV = H * DH + 2 * KVH * DH

    qkv, gate = pl.pallas_call(
        _qkv_kernel,
        grid=(NQT,),
        in_specs=[
            pl.BlockSpec((TQ, D), lambda i: (i, 0)),
            pl.BlockSpec((1, D), lambda i: (0, 0)),
            pl.BlockSpec((D, NQKV), lambda i: (0, 0)),
            pl.BlockSpec((D, 3 * H), lambda i: (0, 0)),
            pl.BlockSpec((1, 3 * H), lambda i: (0, 0)),
        ],
        out_specs=[
            pl.BlockSpec((TQ, NQKV), lambda i: (i, 0)),
            pl.BlockSpec((TQ, 3 * H), lambda i: (i, 0)),
        ],
        out_shape=[
            jax.ShapeDtypeStruct((T, NQKV), jnp.float32),
            jax.ShapeDtypeStruct((T, 3 * H), jnp.float32),
        ],
        interpret=_INTERPRET,
    )(x, gamma.reshape(1, D), Wqkv, Wcomb, bcomb.reshape(1, 3 * H))

    k = jax.lax.slice(qkv, (0, H * DH), (T, H * DH + DH))
    v = jax.lax.slice(qkv, (0, H * DH + DH), (T, H * DH + 2 * DH))
    kw = k.reshape(W, BLK * DH)
    vw = v.reshape(W, BLK * DH)
    vb = v.astype(jnp.bfloat16)
    expandT = (jnp.arange(T)[:, None] // BLK
               == jnp.arange(W)[None, :]).astype(jnp.float32)
    kx = jnp.concatenate([k, expandT], axis=1)
    q4 = qkv[:, :H * DH].reshape(T, H, DH).transpose(1, 0, 2)
    g0, g1, g2 = (gate[:, j::3].T.reshape(H, T, 1) for j in (0, 1, 2))

    full = lambda shape: pl.BlockSpec(shape, lambda: tuple(0 for _ in shape))
    ck, cv = pl.pallas_call(
        _compress_kernel,
        in_specs=[
            full((W, BLK * DH)), full((W, BLK * DH)),
            full((BLK * DH, BLK * DH)), full((1, BLK * DH)),
            full((BLK * DH, DH)), full((1, DH)),
            full((BLK * DH, BLK * DH)), full((1, BLK * DH)),
            full((BLK * DH, DH)), full((1, DH)),
            full((1, DH)), full((1, DH)),
        ],
        out_specs=[full((2 * W, DH)), full((2 * W, DH))],
        out_shape=[
            jax.ShapeDtypeStruct((2 * W, DH), jnp.float32),
            jax.ShapeDtypeStruct((2 * W, DH), jnp.float32),
        ],
        interpret=_INTERPRET,
    )(kw, vw, kW1, kb1.reshape(1, -1), kW2, kb2.reshape(1, -1),
      vW1, vb1.reshape(1, -1), vW2, vb2.reshape(1, -1),
      mem_kv[0, 0], mem_kv[1, 0])

    posp = jnp.pad(pos, ((0, 0), (0, 8 - pos.shape[1])))
    sigma3 = sigma_att.reshape(H, 1, 1)
    out = pl.pallas_call(
        _mega_kernel,
        grid=(NQT,),
        in_specs=[
            pl.BlockSpec((H, TQ, DH), lambda i: (0, i, 0)),
            pl.BlockSpec((2 * W, DH), lambda i: (0, 0)),
            pl.BlockSpec((2 * W, DH), lambda i: (0, 0)),
            pl.BlockSpec((T, DH + W), lambda i: (0, 0)),
            pl.BlockSpec((T, DH), lambda i: (0, 0)),
            pl.BlockSpec((TQ, DH), lambda i: (i, 0)),
            pl.BlockSpec((TQ, DH), lambda i: (i, 0)),
            pl.BlockSpec((TQ, 8), lambda i: (i, 0)),
            pl.BlockSpec((H, 1, 1), lambda i: (0, 0, 0)),
            pl.BlockSpec((H, TQ, 1), lambda i: (0, i, 0)),
            pl.BlockSpec((H, TQ, 1), lambda i: (0, i, 0)),
            pl.BlockSpec((H, TQ, 1), lambda i: (0, i, 0)),
            pl.BlockSpec((H * DH, D), lambda i: (0, 0)),
        ],
        out_specs=pl.BlockSpec((TQ, D), lambda i: (i, 0)),
        out_shape=jax.ShapeDtypeStruct((T, D), jnp.float32),
        interpret=_INTERPRET,
    )(q4, ck, cv.astype(jnp.bfloat16), kx, vb, k, vb, posp, sigma3,
      g0, g1, g2, Wout)

    return out.reshape(B, T, D)


# speedup vs baseline: 5.3421x; 1.0123x over previous
import functools
import jax
import jax.numpy as jnp
from jax import lax
from jax.experimental import pallas as pl
from jax.experimental.pallas import tpu as pltpu
from jax.experimental.pallas import tpu_sc as plsc

B, T, D = 1, 2048, 768
H, KVH, DH = 16, 1, 64
G = H // KVH
BLK = 32
NSEL = 16
BALL = 128
W = T // BLK
NB = T // BALL
SCALE = DH ** -0.5
TQ = 256
NQT = T // TQ
HB = 4
NBT = TQ // BALL
NEG = -jnp.finfo(jnp.float32).max / 10.0
MBIG = 512.0

_INTERPRET = False


def _dotT(a, b):
    return jax.lax.dot_general(a, b, (((1,), (1,)), ((), ())),
                               preferred_element_type=jnp.float32)


def _dot(a, b):
    return jnp.dot(a, b, preferred_element_type=jnp.float32)


def _softmax_unnorm(s):
    m = jnp.max(s, axis=-1, keepdims=True)
    e = jnp.exp(s - m)
    denom = jnp.sum(e, axis=-1, keepdims=True)
    return e.astype(jnp.bfloat16), 1.0 / denom


def _make_sc_mask():
    info = plsc.get_sparse_core_info()
    NC, NS, L = info.num_cores, info.num_subcores, info.num_lanes
    QW = 128
    NWORK = T // QW
    mesh = plsc.VectorSubcoreMesh(core_axis_name="c", subcore_axis_name="s")

    @functools.partial(
        pl.kernel, mesh=mesh,
        out_type=jax.ShapeDtypeStruct((W, T), jnp.float32),
        scratch_types=[
            pltpu.VMEM((W, QW), jnp.float32),
            pltpu.VMEM((W, QW), jnp.float32),
        ],
    )
    def sc_mask(impT_hbm, maskT_hbm, impv, maskv):
        wid = lax.axis_index("s") * NC + lax.axis_index("c")

        @pl.when(wid < NWORK)
        def _():
            base = wid * QW
            pltpu.sync_copy(impT_hbm.at[:, pl.ds(base, QW)], impv)
            for vg in range(QW // L):
                sl = pl.ds(vg * L, L)

                def pass_body(p, carry):
                    t_val, t_idx = carry
                    m_val = jnp.full((L,), -jnp.inf, jnp.float32)
                    m_idx = jnp.full((L,), W, jnp.int32)
                    for j in range(W):
                        x = impv[j, sl]
                        jc = jnp.full((L,), j, jnp.int32)
                        elig = (x < t_val) | ((x == t_val) & (jc > t_idx))
                        better = elig & ((x > m_val)
                                         | ((x == m_val) & (jc < m_idx)))
                        m_val = jnp.where(better, x, m_val)
                        m_idx = jnp.where(better, jc, m_idx)
                    return m_val, m_idx

                t0 = (jnp.full((L,), jnp.inf, jnp.float32),
                      jnp.full((L,), -1, jnp.int32))
                t_val, t_idx = lax.fori_loop(0, NSEL, pass_body, t0)
                for j in range(W):
                    x = impv[j, sl]
                    jc = jnp.full((L,), j, jnp.int32)
                    sel = (x > t_val) | ((x == t_val) & (jc <= t_idx))
                    maskv[j, sl] = jnp.where(sel, jnp.float32(1.0),
                                             jnp.float32(0.0))
            pltpu.sync_copy(maskv, maskT_hbm.at[:, pl.ds(base, QW)])

    return sc_mask


def _qkv_kernel(x_ref, gamma_ref, wqkv_ref, wcomb_ref, bcomb_ref,
                qkv_ref, gate_ref):
    x = x_ref[...]
    eps = jnp.finfo(jnp.float32).eps
    xn = x * jax.lax.rsqrt(jnp.mean(x * x, axis=-1, keepdims=True) + eps)
    xn = xn * gamma_ref[...]
    qkv_ref[...] = _dot(xn, wqkv_ref[...])
    gate_ref[...] = jax.nn.sigmoid(_dot(xn, wcomb_ref[...]) + bcomb_ref[...])


def _compress_kernel(kw_ref, vw_ref, kW1_ref, kb1_ref, kW2_ref, kb2_ref,
                     vW1_ref, vb1_ref, vW2_ref, vb2_ref, memk_ref, memv_ref,
                     ck_ref, cv_ref):
    hk = jnp.maximum(_dot(kw_ref[...], kW1_ref[...]) + kb1_ref[...], 0.0)
    ck = _dot(hk, kW2_ref[...]) + kb2_ref[...]
    hv = jnp.maximum(_dot(vw_ref[...], vW1_ref[...]) + vb1_ref[...], 0.0)
    cv = _dot(hv, vW2_ref[...]) + vb2_ref[...]
    ck_ref[...] = jnp.zeros((2 * W, DH), jnp.float32)
    cv_ref[...] = jnp.zeros((2 * W, DH), jnp.float32)
    ck_ref[0:W, :] = ck
    cv_ref[0:W, :] = cv
    ck_ref[W:W + 1, :] = memk_ref[...]
    cv_ref[W:W + 1, :] = memv_ref[...]


def _cattn_kernel(q_ref, ck_ref, cv_ref, g_ref, co_ref, impT_ref):
    q = q_ref[...].reshape(H * TQ, DH)
    s = _dotT(q, ck_ref[...]) * SCALE
    col = jax.lax.broadcasted_iota(jnp.int32, s.shape, 1)
    sm = jnp.where(col <= W, s, NEG)
    e, rdenom = _softmax_unnorm(sm)
    co = (_dot(e, cv_ref[...]) * rdenom * g_ref[...].reshape(H * TQ, 1)
          ).reshape(H, TQ, DH)
    co_ref[...] = co
    imp = jnp.mean(s.reshape(H, TQ, 2 * W)[:, :, 0:W], axis=0)
    impT_ref[...] = jnp.transpose(imp)


def _mega2_kernel(q_ref, maskT_ref, co_ref, kx_ref, vb_ref, kloc_ref,
                  vloc_ref, pos_ref, sigma_ref, g1_ref, g2_ref, wout_ref,
                  out_ref):
    q_all = q_ref[...]
    mtile = jnp.transpose(maskT_ref[...]) * MBIG

    pos = pos_ref[...]
    g2 = g2_ref[...]
    ball_outs = []
    for bi in range(NBT):
        pb = pos[bi * BALL:(bi + 1) * BALL]
        pt = jnp.transpose(pb)
        d2 = jnp.zeros((BALL, BALL), jnp.float32)
        for ci in range(3):
            diff = pb[:, ci:ci + 1] - pt[ci:ci + 1, :]
            d2 = d2 + diff * diff
        dist = jnp.sqrt(jnp.maximum(d2, 0.0))
        sig = jnp.broadcast_to(sigma_ref[...], (H, BALL, 1)
                               ).reshape(H * BALL, 1)
        bias = sig * jnp.broadcast_to(dist[None], (H, BALL, BALL)
                                      ).reshape(H * BALL, BALL)
        qb = q_all[:, bi * BALL:(bi + 1) * BALL, :].reshape(H * BALL, DH)
        sb = _dotT(qb * SCALE, kloc_ref[bi * BALL:(bi + 1) * BALL, :]) + bias
        eb, rdb = _softmax_unnorm(sb)
        ob = _dot(eb, vloc_ref[bi * BALL:(bi + 1) * BALL, :]) * rdb
        ob = ob * g2[:, bi * BALL:(bi + 1) * BALL, :].reshape(H * BALL, 1)
        ball_outs.append(ob.reshape(H, BALL, DH))
    so = jnp.concatenate(ball_outs, axis=1)

    g1 = g1_ref[...]
    co = co_ref[...]
    acc = jnp.zeros((TQ, D), jnp.float32)
    for hg in range(H // HB):
        qh = q_all[hg * HB:(hg + 1) * HB].reshape(HB * TQ, DH) * SCALE
        qa = jnp.concatenate(
            [qh, jnp.broadcast_to(mtile[None], (HB, TQ, W)
                                  ).reshape(HB * TQ, W)], axis=1)
        sf = _dotT(qa, kx_ref[...])
        ef, rdf = _softmax_unnorm(sf.reshape(HB, TQ, T))
        fine = _dot(ef.reshape(HB * TQ, T), vb_ref[...]).reshape(HB, TQ, DH)
        mixed = (fine * rdf * g1[hg * HB:(hg + 1) * HB]
                 + co[hg * HB:(hg + 1) * HB] + so[hg * HB:(hg + 1) * HB])
        for j in range(HB):
            hh = hg * HB + j
            acc = acc + _dot(mixed[j], wout_ref[hh * DH:(hh + 1) * DH, :])
    out_ref[...] = acc


def kernel(inp, pos, gamma, Wqkv, mem_kv, kW1, kb1, kW2, kb2,
           vW1, vb1, vW2, vb2, sigma_att, Wcomb, bcomb, Wout):
    x = inp.reshape(T, D)
    NQKV = H * DH + 2 * KVH * DH

    qkv, gate = pl.pallas_call(
        _qkv_kernel,
        grid=(NQT,),
        in_specs=[
            pl.BlockSpec((TQ, D), lambda i: (i, 0)),
            pl.BlockSpec((1, D), lambda i: (0, 0)),
            pl.BlockSpec((D, NQKV), lambda i: (0, 0)),
            pl.BlockSpec((D, 3 * H), lambda i: (0, 0)),
            pl.BlockSpec((1, 3 * H), lambda i: (0, 0)),
        ],
        out_specs=[
            pl.BlockSpec((TQ, NQKV), lambda i: (i, 0)),
            pl.BlockSpec((TQ, 3 * H), lambda i: (i, 0)),
        ],
        out_shape=[
            jax.ShapeDtypeStruct((T, NQKV), jnp.float32),
            jax.ShapeDtypeStruct((T, 3 * H), jnp.float32),
        ],
        interpret=_INTERPRET,
    )(x, gamma.reshape(1, D), Wqkv, Wcomb, bcomb.reshape(1, 3 * H))

    k = jax.lax.slice(qkv, (0, H * DH), (T, H * DH + DH))
    v = jax.lax.slice(qkv, (0, H * DH + DH), (T, H * DH + 2 * DH))
    kw = k.reshape(W, BLK * DH)
    vw = v.reshape(W, BLK * DH)
    vb = v.astype(jnp.bfloat16)
    expandT = (jnp.arange(T)[:, None] // BLK
               == jnp.arange(W)[None, :]).astype(jnp.float32)
    kx = jnp.concatenate([k, expandT], axis=1)
    q4 = qkv[:, :H * DH].reshape(T, H, DH).transpose(1, 0, 2)
    g0, g1, g2 = (gate[:, j::3].T.reshape(H, T, 1) for j in (0, 1, 2))

    full = lambda shape: pl.BlockSpec(shape, lambda: tuple(0 for _ in shape))
    ck, cv = pl.pallas_call(
        _compress_kernel,
        in_specs=[
            full((W, BLK * DH)), full((W, BLK * DH)),
            full((BLK * DH, BLK * DH)), full((1, BLK * DH)),
            full((BLK * DH, DH)), full((1, DH)),
            full((BLK * DH, BLK * DH)), full((1, BLK * DH)),
            full((BLK * DH, DH)), full((1, DH)),
            full((1, DH)), full((1, DH)),
        ],
        out_specs=[full((2 * W, DH)), full((2 * W, DH))],
        out_shape=[
            jax.ShapeDtypeStruct((2 * W, DH), jnp.float32),
            jax.ShapeDtypeStruct((2 * W, DH), jnp.float32),
        ],
        interpret=_INTERPRET,
    )(kw, vw, kW1, kb1.reshape(1, -1), kW2, kb2.reshape(1, -1),
      vW1, vb1.reshape(1, -1), vW2, vb2.reshape(1, -1),
      mem_kv[0, 0], mem_kv[1, 0])

    co4, impT = pl.pallas_call(
        _cattn_kernel,
        grid=(NQT,),
        in_specs=[
            pl.BlockSpec((H, TQ, DH), lambda i: (0, i, 0)),
            pl.BlockSpec((2 * W, DH), lambda i: (0, 0)),
            pl.BlockSpec((2 * W, DH), lambda i: (0, 0)),
            pl.BlockSpec((H, TQ, 1), lambda i: (0, i, 0)),
        ],
        out_specs=[
            pl.BlockSpec((H, TQ, DH), lambda i: (0, i, 0)),
            pl.BlockSpec((W, TQ), lambda i: (0, i)),
        ],
        out_shape=[
            jax.ShapeDtypeStruct((H, T, DH), jnp.float32),
            jax.ShapeDtypeStruct((W, T), jnp.float32),
        ],
        interpret=_INTERPRET,
    )(q4, ck, cv.astype(jnp.bfloat16), g0)

    maskT = _make_sc_mask()(impT)

    posp = jnp.pad(pos, ((0, 0), (0, 8 - pos.shape[1])))
    sigma3 = sigma_att.reshape(H, 1, 1)
    out = pl.pallas_call(
        _mega2_kernel,
        grid=(NQT,),
        in_specs=[
            pl.BlockSpec((H, TQ, DH), lambda i: (0, i, 0)),
            pl.BlockSpec((W, TQ), lambda i: (0, i)),
            pl.BlockSpec((H, TQ, DH), lambda i: (0, i, 0)),
            pl.BlockSpec((T, DH + W), lambda i: (0, 0)),
            pl.BlockSpec((T, DH), lambda i: (0, 0)),
            pl.BlockSpec((TQ, DH), lambda i: (i, 0)),
            pl.BlockSpec((TQ, DH), lambda i: (i, 0)),
            pl.BlockSpec((TQ, 8), lambda i: (i, 0)),
            pl.BlockSpec((H, 1, 1), lambda i: (0, 0, 0)),
            pl.BlockSpec((H, TQ, 1), lambda i: (0, i, 0)),
            pl.BlockSpec((H, TQ, 1), lambda i: (0, i, 0)),
            pl.BlockSpec((H * DH, D), lambda i: (0, 0)),
        ],
        out_specs=pl.BlockSpec((TQ, D), lambda i: (i, 0)),
        out_shape=jax.ShapeDtypeStruct((T, D), jnp.float32),
        interpret=_INTERPRET,
    )(q4, maskT, co4, kx, vb, k, vb, posp, sigma3, g1, g2, Wout)

    return out.reshape(B, T, D)
